# baseline (speedup 1.0000x reference)
import functools

import jax
import jax.numpy as jnp
from jax import lax
from jax.experimental import pallas as pl
from jax.experimental.pallas import tpu as pltpu
from jax.experimental.pallas import tpu_sc as plsc

N = 10000
NP = 10240
E = 320000
D = 128
R = 16

NC = 2
NS = 16
HN = NP // 2
ACC_ROWS = HN + 8
ETC = E // NS
B = 32
NST = 5
SB = ETC // (B * NST)
K = 5
SL = HN // NS
ZR = 40


def _sc_pass_body(table_a, table_b, flag, gidx, wgt, dstl, out,
                  flag_v, idx2, dst2, w2, rows, zero_v, acc, gsem, ssem):
    c = lax.axis_index("c")
    s = lax.axis_index("s")

    pltpu.sync_copy(flag, flag_v)
    fv = flag_v[...][0]

    def zrow(i, _):
        for j in range(D // 16):
            zero_v[i, pl.ds(j * 16, 16)] = jnp.zeros((16,), jnp.float32)
        return 0
    lax.fori_loop(0, ZR, zrow, 0)

    def zslab(t, _):
        pltpu.sync_copy(zero_v, acc.at[pl.ds(s * SL + t * ZR, ZR)])
        return 0
    lax.fori_loop(0, SL // ZR, zslab, 0)

    @pl.when(s == 0)
    def _():
        pltpu.sync_copy(zero_v.at[pl.ds(0, 8)], acc.at[pl.ds(HN, 8)])
    plsc.subcore_barrier()

    def run(table, weighted):
        def stage(t, _):
            pltpu.sync_copy(gidx.at[c, s, t], idx2)
            pltpu.sync_copy(dstl.at[c, s, t], dst2)
            if weighted:
                pltpu.sync_copy(wgt.at[c, s, t], w2)

            for b in range(K):
                pltpu.async_copy(table.at[idx2.at[b]], rows.at[b], gsem)

            def rnd(k, _):
                for b in range(K):
                    i = k * K + b
                    pltpu.make_async_copy(table.at[idx2.at[b]], rows.at[b],
                                          gsem).wait()
                    if weighted:
                        def wgrp(g, _):
                            wv16 = w2[i, pl.ds(g * 16, 16)]
                            for l in range(16):
                                wl = wv16[l]
                                e = g * 16 + l
                                for j in range(D // 16):
                                    rows[b, e, pl.ds(j * 16, 16)] = (
                                        rows[b, e, pl.ds(j * 16, 16)] * wl)
                            return 0
                        lax.fori_loop(0, B // 16, wgrp, 0)
                    pltpu.async_copy(rows.at[b], acc.at[dst2.at[i]], ssem,
                                     add=True)
                    pb = (b - 1) % K
                    refill = i + K - 1

                    @pl.when((i >= 1) & (refill < SB))
                    def _():
                        pltpu.make_async_copy(rows.at[pb], acc.at[dst2.at[0]],
                                              ssem).wait()
                        pltpu.async_copy(table.at[idx2.at[refill]],
                                         rows.at[pb], gsem)
                return 0
            lax.fori_loop(0, SB // K, rnd, 0)

            for b in range(K):
                pltpu.make_async_copy(rows.at[b], acc.at[dst2.at[0]],
                                      ssem).wait()
            return 0
        lax.fori_loop(0, NST, stage, 0)

    lax.cond(fv == 0,
             lambda: run(table_a, True),
             lambda: run(table_b, False))

    plsc.subcore_barrier()
    pltpu.sync_copy(acc.at[pl.ds(s * SL, SL)], out.at[pl.ds(c * HN + s * SL, SL)])


@functools.lru_cache(maxsize=None)
def _make_sc_pass():
    mesh = plsc.VectorSubcoreMesh(core_axis_name="c", subcore_axis_name="s",
                                  num_cores=NC, num_subcores=NS)
    scratch = [
        pltpu.VMEM((16,), jnp.int32),
        pltpu.VMEM((SB, B), jnp.int32),
        pltpu.VMEM((SB, B), jnp.int32),
        pltpu.VMEM((SB, B), jnp.float32),
        pltpu.VMEM((K, B, D), jnp.float32),
        pltpu.VMEM((ZR, D), jnp.float32),
        pltpu.VMEM_SHARED((ACC_ROWS, D), jnp.float32),
        pltpu.SemaphoreType.DMA,
        pltpu.SemaphoreType.DMA,
    ]
    return pl.kernel(
        _sc_pass_body,
        out_type=jax.ShapeDtypeStruct((NP, D), jnp.float32),
        mesh=mesh,
        scratch_types=scratch,
    )


def _sc_pass(table_a, table_b, flag, gidx, wgt, dstl):
    return _make_sc_pass()(table_a, table_b, flag, gidx, wgt, dstl)


BN = 512


def _mm_rel_body(x_ref, w_ref, o_ref):
    o_ref[0] = jnp.dot(x_ref[...], w_ref[0], preferred_element_type=jnp.float32)


def _mm_rel(x_pad, w_rel):
    return pl.pallas_call(
        _mm_rel_body,
        grid=(R, NP // BN),
        in_specs=[
            pl.BlockSpec((BN, D), lambda r, n: (n, 0)),
            pl.BlockSpec((1, D, D), lambda r, n: (r, 0, 0)),
        ],
        out_specs=pl.BlockSpec((1, BN, D), lambda r, n: (r, n, 0)),
        out_shape=jax.ShapeDtypeStruct((R, NP, D), jnp.float32),
    )(x_pad, w_rel)


def _root_add_body(x_ref, w_ref, p_ref, o_ref):
    o_ref[...] = (jnp.dot(x_ref[...], w_ref[...], preferred_element_type=jnp.float32)
                  + p_ref[...])


def _root_add(x_pad, w_root, part):
    return pl.pallas_call(
        _root_add_body,
        grid=(NP // BN,),
        in_specs=[
            pl.BlockSpec((BN, D), lambda n: (n, 0)),
            pl.BlockSpec((D, D), lambda n: (0, 0)),
            pl.BlockSpec((BN, D), lambda n: (n, 0)),
        ],
        out_specs=pl.BlockSpec((BN, D), lambda n: (n, 0)),
        out_shape=jax.ShapeDtypeStruct((NP, D), jnp.float32),
    )(x_pad, w_root, part)


def _halved(vals, fill, dst):
    in0 = dst < HN
    both = jnp.concatenate([jnp.where(in0, vals, fill),
                            jnp.where(in0, fill, vals)])
    return both.reshape(NC, NS, NST, SB, B)


def kernel(edge_index, edge_type, node_emb, rel_w, root_w):
    src, dst = edge_index[0], edge_index[1]

    flag_conv = jnp.zeros((16,), jnp.int32)
    flag_b = jnp.ones((16,), jnp.int32)
    zeros_2e = jnp.zeros((NC, NS, NST, SB, B), jnp.float32)
    eye_pad = jnp.zeros((NP, D), jnp.float32).at[:16, :16].set(
        jnp.eye(16, dtype=jnp.float32))

    in0 = dst < HN
    dstl = jnp.concatenate([jnp.where(in0, dst, HN),
                            jnp.where(in0, HN, dst - HN)]
                           ).reshape(NC, NS, NST, SB, B)
    gidx_cnt = _halved(edge_type, N, dst)
    gidx_ppv = _halved(src, N, dst)

    xp = jnp.zeros((NP, D), jnp.float32).at[:N].set(node_emb)
    y0 = _mm_rel(xp, rel_w[0]).reshape(R * NP, D)

    cnt = _sc_pass(y0, eye_pad, flag_b, gidx_cnt, zeros_2e, dstl)
    cnt16 = cnt[:, :16]
    inv_tot = 1.0 / jnp.clip(jnp.sum(cnt16, axis=1), 1.0)
    w_conv = 1.0 / jnp.clip(cnt16[dst, edge_type], 1.0)
    wgt_conv = _halved(w_conv, 0.0, dst)
    gidx_conv = _halved(edge_type * NP + src, N, dst)

    def conv_from_y(x_pad, y, r_l):
        part = _sc_pass(y, eye_pad, flag_conv, gidx_conv, wgt_conv, dstl)
        return _root_add(x_pad, r_l, part)

    def conv(x_pad, w_l, r_l):
        y = _mm_rel(x_pad, w_l).reshape(R * NP, D)
        return conv_from_y(x_pad, y, r_l)

    def ppv(x_pad, y_any):
        pos = (x_pad > 0).astype(jnp.float32)
        sums = _sc_pass(y_any, pos, flag_b, gidx_ppv, zeros_2e, dstl)
        return sums * inv_tot[:, None]

    x1 = conv_from_y(xp, y0, root_w[0])
    ppv1 = ppv(x1, y0)
    x2 = conv(jax.nn.relu(x1), rel_w[1], root_w[1])
    p2 = conv(ppv1, rel_w[1], root_w[1])
    ppv2 = ppv(p2, y0)
    return jnp.concatenate([x2[:N], ppv2[:N]], axis=1)


# speedup vs baseline: 24.8275x; 24.8275x over previous
import functools

import jax
import jax.numpy as jnp
from jax import lax
from jax.experimental import pallas as pl
from jax.experimental.pallas import tpu as pltpu
from jax.experimental.pallas import tpu_sc as plsc

N = 10000
NP = 10240
E = 320000
D = 128
R = 16

NC = 2
NS = 16
HN = NP // 2
ACC_ROWS = HN + 128
ETC = E // NS
B = 32
NST = 5
SB = ETC // (B * NST)
K = 5
SL = HN // NS
ZR = 40


def _sc_pass_body(table_a, table_b, flag, gidx, wgt, dstl, out,
                  flag_v, idx2, dst2, w2, rows, zero_v, acc, gsem, ssem):
    c = lax.axis_index("c")
    s = lax.axis_index("s")

    pltpu.sync_copy(flag, flag_v)
    fv = flag_v[...][0]

    def zrow(i, _):
        for j in range(D // 16):
            zero_v[i, pl.ds(j * 16, 16)] = jnp.zeros((16,), jnp.float32)
        return 0
    lax.fori_loop(0, ZR, zrow, 0)

    def zslab(t, _):
        pltpu.sync_copy(zero_v, acc.at[pl.ds(s * SL + t * ZR, ZR)])
        return 0
    lax.fori_loop(0, SL // ZR, zslab, 0)

    pltpu.sync_copy(zero_v.at[pl.ds(0, 8)], acc.at[pl.ds(HN + s * 8, 8)])
    plsc.subcore_barrier()

    def run(table, weighted):
        def stage(t, _):
            pltpu.sync_copy(gidx.at[c, s, t], idx2)
            pltpu.sync_copy(dstl.at[c, s, t], dst2)
            if weighted:
                pltpu.sync_copy(wgt.at[c, s, t], w2)

            for b in range(K):
                pltpu.async_copy(table.at[idx2.at[b]], rows.at[b], gsem)

            def rnd(k, _):
                for b in range(K):
                    i = k * K + b
                    pltpu.make_async_copy(table.at[idx2.at[b]], rows.at[b],
                                          gsem).wait()
                    if weighted:
                        def wgrp(g, _):
                            wv16 = w2[i, pl.ds(g * 16, 16)]
                            for l in range(16):
                                wl = wv16[l]
                                e = g * 16 + l
                                for j in range(D // 16):
                                    rows[b, e, pl.ds(j * 16, 16)] = (
                                        rows[b, e, pl.ds(j * 16, 16)] * wl)
                            return 0
                        lax.fori_loop(0, B // 16, wgrp, 0)
                    pltpu.async_copy(rows.at[b], acc.at[dst2.at[i]], ssem,
                                     add=True)
                    pb = (b - 1) % K
                    refill = i + K - 1

                    @pl.when((i >= 1) & (refill < SB))
                    def _():
                        pltpu.make_async_copy(rows.at[pb], acc.at[dst2.at[0]],
                                              ssem).wait()
                        pltpu.async_copy(table.at[idx2.at[refill]],
                                         rows.at[pb], gsem)
                return 0
            lax.fori_loop(0, SB // K, rnd, 0)

            for b in range(K):
                pltpu.make_async_copy(rows.at[b], acc.at[dst2.at[0]],
                                      ssem).wait()
            return 0
        lax.fori_loop(0, NST, stage, 0)

    lax.cond(fv == 0,
             lambda: run(table_a, True),
             lambda: run(table_b, False))

    plsc.subcore_barrier()
    pltpu.sync_copy(acc.at[pl.ds(s * SL, SL)], out.at[pl.ds(c * HN + s * SL, SL)])


@functools.lru_cache(maxsize=None)
def _make_sc_pass():
    mesh = plsc.VectorSubcoreMesh(core_axis_name="c", subcore_axis_name="s",
                                  num_cores=NC, num_subcores=NS)
    scratch = [
        pltpu.VMEM((16,), jnp.int32),
        pltpu.VMEM((SB, B), jnp.int32),
        pltpu.VMEM((SB, B), jnp.int32),
        pltpu.VMEM((SB, B), jnp.float32),
        pltpu.VMEM((K, B, D), jnp.float32),
        pltpu.VMEM((ZR, D), jnp.float32),
        pltpu.VMEM_SHARED((ACC_ROWS, D), jnp.float32),
        pltpu.SemaphoreType.DMA,
        pltpu.SemaphoreType.DMA,
    ]
    return pl.kernel(
        _sc_pass_body,
        out_type=jax.ShapeDtypeStruct((NP, D), jnp.float32),
        mesh=mesh,
        scratch_types=scratch,
    )


def _sc_pass(table_a, table_b, flag, gidx, wgt, dstl):
    return _make_sc_pass()(table_a, table_b, flag, gidx, wgt, dstl)


BN = 512


def _mm_rel_body(x_ref, w_ref, o_ref):
    o_ref[0] = jnp.dot(x_ref[...], w_ref[0], preferred_element_type=jnp.float32)


def _mm_rel(x_pad, w_rel):
    return pl.pallas_call(
        _mm_rel_body,
        grid=(R, NP // BN),
        in_specs=[
            pl.BlockSpec((BN, D), lambda r, n: (n, 0)),
            pl.BlockSpec((1, D, D), lambda r, n: (r, 0, 0)),
        ],
        out_specs=pl.BlockSpec((1, BN, D), lambda r, n: (r, n, 0)),
        out_shape=jax.ShapeDtypeStruct((R, NP, D), jnp.float32),
    )(x_pad, w_rel)


def _root_add_body(x_ref, w_ref, p_ref, o_ref):
    o_ref[...] = (jnp.dot(x_ref[...], w_ref[...], preferred_element_type=jnp.float32)
                  + p_ref[...])


def _root_add(x_pad, w_root, part):
    return pl.pallas_call(
        _root_add_body,
        grid=(NP // BN,),
        in_specs=[
            pl.BlockSpec((BN, D), lambda n: (n, 0)),
            pl.BlockSpec((D, D), lambda n: (0, 0)),
            pl.BlockSpec((BN, D), lambda n: (n, 0)),
        ],
        out_specs=pl.BlockSpec((BN, D), lambda n: (n, 0)),
        out_shape=jax.ShapeDtypeStruct((NP, D), jnp.float32),
    )(x_pad, w_root, part)


def _halved(vals, fill, dst):
    in0 = dst < HN
    both = jnp.concatenate([jnp.where(in0, vals, fill),
                            jnp.where(in0, fill, vals)])
    return both.reshape(NC, NS, NST, SB, B)


def kernel(edge_index, edge_type, node_emb, rel_w, root_w):
    src, dst = edge_index[0], edge_index[1]

    flag_conv = jnp.zeros((16,), jnp.int32)
    flag_b = jnp.ones((16,), jnp.int32)
    zeros_2e = jnp.zeros((NC, NS, NST, SB, B), jnp.float32)
    eye_pad = jnp.zeros((NP, D), jnp.float32).at[:16, :16].set(
        jnp.eye(16, dtype=jnp.float32))

    in0 = dst < HN
    eidx = jnp.arange(E, dtype=jnp.int32)
    zrow = N + eidx % (NP - N)
    drow = HN + eidx % 128
    dstl = jnp.concatenate([jnp.where(in0, dst, drow),
                            jnp.where(in0, drow, dst - HN)]
                           ).reshape(NC, NS, NST, SB, B)
    gidx_cnt = _halved(edge_type, zrow, dst)
    gidx_ppv = _halved(src, zrow, dst)

    xp = jnp.zeros((NP, D), jnp.float32).at[:N].set(node_emb)
    y0 = _mm_rel(xp, rel_w[0]).reshape(R * NP, D)

    cnt = _sc_pass(y0, eye_pad, flag_b, gidx_cnt, zeros_2e, dstl)
    cnt16 = cnt[:, :16]
    inv_tot = 1.0 / jnp.clip(jnp.sum(cnt16, axis=1), 1.0)
    w_conv = 1.0 / jnp.clip(cnt16[dst, edge_type], 1.0)
    wgt_conv = _halved(w_conv, 0.0, dst)
    gidx_conv = _halved(edge_type * NP + src, zrow, dst)

    def conv_from_y(x_pad, y, r_l):
        part = _sc_pass(y, eye_pad, flag_conv, gidx_conv, wgt_conv, dstl)
        return _root_add(x_pad, r_l, part)

    def conv(x_pad, w_l, r_l):
        y = _mm_rel(x_pad, w_l).reshape(R * NP, D)
        return conv_from_y(x_pad, y, r_l)

    def ppv(x_pad, y_any):
        pos = (x_pad > 0).astype(jnp.float32)
        sums = _sc_pass(y_any, pos, flag_b, gidx_ppv, zeros_2e, dstl)
        return sums * inv_tot[:, None]

    x1 = conv_from_y(xp, y0, root_w[0])
    ppv1 = ppv(x1, y0)
    x2 = conv(jax.nn.relu(x1), rel_w[1], root_w[1])
    p2 = conv(ppv1, rel_w[1], root_w[1])
    ppv2 = ppv(p2, y0)
    return jnp.concatenate([x2[:N], ppv2[:N]], axis=1)


# speedup vs baseline: 33.0912x; 1.3328x over previous
import functools

import jax
import jax.numpy as jnp
from jax import lax
from jax.experimental import pallas as pl
from jax.experimental.pallas import tpu as pltpu
from jax.experimental.pallas import tpu_sc as plsc

N = 10000
NP = 10240
E = 320000
D = 128
R = 16

NC = 2
NS = 16
HN = NP // 2
ACC_ROWS = HN + 128
ETC = E // NS
B = 32
NST = 5
SB = ETC // (B * NST)
K = 5
SL = HN // NS
ZR = 40


def _sc_pass_body(table_a, table_b, flag, gidx, wgt, dstl, out,
                  flag_v, idx2, dst2, w2, rows, zero_v, acc, gsem, ssem):
    c = lax.axis_index("c")
    s = lax.axis_index("s")

    pltpu.sync_copy(flag, flag_v)
    fv = flag_v[...][0]

    def zrow(i, _):
        for j in range(D // 16):
            zero_v[i, pl.ds(j * 16, 16)] = jnp.zeros((16,), jnp.float32)
        return 0
    lax.fori_loop(0, ZR, zrow, 0)

    def zslab(t, _):
        pltpu.sync_copy(zero_v, acc.at[pl.ds(s * SL + t * ZR, ZR)])
        return 0
    lax.fori_loop(0, SL // ZR, zslab, 0)

    pltpu.sync_copy(zero_v.at[pl.ds(0, 8)], acc.at[pl.ds(HN + s * 8, 8)])
    plsc.subcore_barrier()

    def run(table, weighted):
        def stage(t, _):
            pltpu.sync_copy(gidx.at[c, s, t], idx2)
            pltpu.sync_copy(dstl.at[c, s, t], dst2)
            if weighted:
                pltpu.sync_copy(wgt.at[c, s, t], w2)

            for b in range(K):
                pltpu.async_copy(table.at[idx2.at[b]], rows.at[b], gsem)

            def rnd(k, _):
                for b in range(K):
                    i = k * K + b
                    pltpu.make_async_copy(table.at[idx2.at[b]], rows.at[b],
                                          gsem).wait()
                    if weighted:
                        def wgrp(g, _):
                            wv16 = w2[i, pl.ds(g * 16, 16)]
                            for l in range(16):
                                wl = wv16[l]
                                e = g * 16 + l
                                for j in range(D // 16):
                                    rows[b, e, pl.ds(j * 16, 16)] = (
                                        rows[b, e, pl.ds(j * 16, 16)] * wl)
                            return 0
                        lax.fori_loop(0, B // 16, wgrp, 0)
                    pltpu.async_copy(rows.at[b], acc.at[dst2.at[i]], ssem,
                                     add=True)
                    pb = (b - 1) % K
                    refill = i + K - 1

                    @pl.when((i >= 1) & (refill < SB))
                    def _():
                        pltpu.make_async_copy(rows.at[pb], acc.at[dst2.at[0]],
                                              ssem).wait()
                        pltpu.async_copy(table.at[idx2.at[refill]],
                                         rows.at[pb], gsem)
                return 0
            lax.fori_loop(0, SB // K, rnd, 0)

            for b in range(K):
                pltpu.make_async_copy(rows.at[b], acc.at[dst2.at[0]],
                                      ssem).wait()
            return 0
        lax.fori_loop(0, NST, stage, 0)

    lax.cond(fv == 0,
             lambda: run(table_a, True),
             lambda: run(table_b, False))

    plsc.subcore_barrier()
    pltpu.sync_copy(acc.at[pl.ds(s * SL, SL)], out.at[pl.ds(c * HN + s * SL, SL)])


@functools.lru_cache(maxsize=None)
def _make_sc_pass():
    mesh = plsc.VectorSubcoreMesh(core_axis_name="c", subcore_axis_name="s",
                                  num_cores=NC, num_subcores=NS)
    scratch = [
        pltpu.VMEM((16,), jnp.int32),
        pltpu.VMEM((SB, B), jnp.int32),
        pltpu.VMEM((SB, B), jnp.int32),
        pltpu.VMEM((SB, B), jnp.float32),
        pltpu.VMEM((K, B, D), jnp.float32),
        pltpu.VMEM((ZR, D), jnp.float32),
        pltpu.VMEM_SHARED((ACC_ROWS, D), jnp.float32),
        pltpu.SemaphoreType.DMA,
        pltpu.SemaphoreType.DMA,
    ]
    return pl.kernel(
        _sc_pass_body,
        out_type=jax.ShapeDtypeStruct((NP, D), jnp.float32),
        mesh=mesh,
        scratch_types=scratch,
    )


def _sc_pass(table_a, table_b, flag, gidx, wgt, dstl):
    return _make_sc_pass()(table_a, table_b, flag, gidx, wgt, dstl)


BN = 512


def _mm_rel_body(x_ref, w_ref, o_ref):
    o_ref[0] = jnp.dot(x_ref[...], w_ref[0], preferred_element_type=jnp.float32)


def _mm_rel(x_pad, w_rel):
    return pl.pallas_call(
        _mm_rel_body,
        grid=(R, NP // BN),
        in_specs=[
            pl.BlockSpec((BN, D), lambda r, n: (n, 0)),
            pl.BlockSpec((1, D, D), lambda r, n: (r, 0, 0)),
        ],
        out_specs=pl.BlockSpec((1, BN, D), lambda r, n: (r, n, 0)),
        out_shape=jax.ShapeDtypeStruct((R, NP, D), jnp.float32),
    )(x_pad, w_rel)


def _root_add_body(x_ref, w_ref, p_ref, o_ref):
    o_ref[...] = (jnp.dot(x_ref[...], w_ref[...], preferred_element_type=jnp.float32)
                  + p_ref[...])


def _root_add(x_pad, w_root, part):
    return pl.pallas_call(
        _root_add_body,
        grid=(NP // BN,),
        in_specs=[
            pl.BlockSpec((BN, D), lambda n: (n, 0)),
            pl.BlockSpec((D, D), lambda n: (0, 0)),
            pl.BlockSpec((BN, D), lambda n: (n, 0)),
        ],
        out_specs=pl.BlockSpec((BN, D), lambda n: (n, 0)),
        out_shape=jax.ShapeDtypeStruct((NP, D), jnp.float32),
    )(x_pad, w_root, part)


def _halved(vals, fill, dst):
    in0 = dst < HN
    both = jnp.concatenate([jnp.where(in0, vals, fill),
                            jnp.where(in0, fill, vals)])
    return both.reshape(NC, NS, NST, SB, B)


def kernel(edge_index, edge_type, node_emb, rel_w, root_w):
    src, dst = edge_index[0], edge_index[1]

    flag_conv = jnp.zeros((16,), jnp.int32)
    flag_b = jnp.ones((16,), jnp.int32)
    zeros_2e = jnp.zeros((NC, NS, NST, SB, B), jnp.float32)
    rr = jnp.arange(NP, dtype=jnp.int32)
    eye_pad = ((rr[:, None] % 16 == jnp.arange(D, dtype=jnp.int32)[None, :])
               & (rr[:, None] < 9600)).astype(jnp.float32)

    in0 = dst < HN
    eidx = jnp.arange(E, dtype=jnp.int32)
    zrow = N + eidx % (NP - N)
    drow = HN + eidx % 128
    dstl = jnp.concatenate([jnp.where(in0, dst, drow),
                            jnp.where(in0, drow, dst - HN)]
                           ).reshape(NC, NS, NST, SB, B)
    gidx_cnt = _halved(edge_type + 16 * (eidx % 600), zrow, dst)
    gidx_ppv = _halved(src, zrow, dst)

    xp = jnp.zeros((NP, D), jnp.float32).at[:N].set(node_emb)
    y0 = _mm_rel(xp, rel_w[0]).reshape(R * NP, D)

    cnt = _sc_pass(y0, eye_pad, flag_b, gidx_cnt, zeros_2e, dstl)
    cnt16 = cnt[:, :16]
    inv_tot = 1.0 / jnp.clip(jnp.sum(cnt16, axis=1), 1.0)
    w_conv = 1.0 / jnp.clip(cnt16[dst, edge_type], 1.0)
    wgt_conv = _halved(w_conv, 0.0, dst)
    gidx_conv = _halved(edge_type * NP + src, zrow, dst)

    def conv_from_y(x_pad, y, r_l):
        part = _sc_pass(y, eye_pad, flag_conv, gidx_conv, wgt_conv, dstl)
        return _root_add(x_pad, r_l, part)

    def conv(x_pad, w_l, r_l):
        y = _mm_rel(x_pad, w_l).reshape(R * NP, D)
        return conv_from_y(x_pad, y, r_l)

    def ppv(x_pad, y_any):
        pos = (x_pad > 0).astype(jnp.float32)
        sums = _sc_pass(y_any, pos, flag_b, gidx_ppv, zeros_2e, dstl)
        return sums * inv_tot[:, None]

    x1 = conv_from_y(xp, y0, root_w[0])
    ppv1 = ppv(x1, y0)
    x2 = conv(jax.nn.relu(x1), rel_w[1], root_w[1])
    p2 = conv(ppv1, rel_w[1], root_w[1])
    ppv2 = ppv(p2, y0)
    return jnp.concatenate([x2[:N], ppv2[:N]], axis=1)
